# Initial kernel scaffold; baseline (speedup 1.0000x reference)
import jax
import jax.numpy as jnp
from jax.experimental import pallas as pl


def kernel(x, edge_index, W1, b1, W2, b2, W3, b3, W4, b4, g1, be1, g2, be2, g3, be3):
    raise NotImplementedError("write your pallas kernel here")


# speedup vs baseline: 16.8136x; 16.8136x over previous
import functools
import math

import jax
import jax.numpy as jnp
from jax import lax
from jax.experimental import pallas as pl
from jax.experimental.pallas import tpu as pltpu
from jax.experimental.pallas import tpu_sc as plsc

N = 10000
NP = 10240
E = 320000
G = 128
NG = 2560
E2 = NG * G
NC = 2
NS = 16
NW = NC * NS
GPW = NG // NW
K = 4
NCHUNK = GPW // K
RPT = NP // NS
BN = 1024
NBLK = NP // BN
EPS = 1e-5
BNS = float(1.0 / math.sqrt(1.0 + EPS))


def _elu(v):
    return jnp.where(v > 0, v, jnp.exp(jnp.minimum(v, 0.0)) - 1.0)


def _mesh():
    return plsc.VectorSubcoreMesh(core_axis_name="c", subcore_axis_name="s")


_SC_PARAMS = pltpu.CompilerParams(use_tc_tiling_on_sc=False)


def _make_agg(F):
    @functools.partial(
        pl.kernel,
        out_type=(jax.ShapeDtypeStruct((NP, F), jnp.float32),
                  jax.ShapeDtypeStruct((NP, F), jnp.float32)),
        mesh=_mesh(),
        compiler_params=_SC_PARAMS,
        scratch_types=[
            pltpu.VMEM((K, G), jnp.int32),
            pltpu.VMEM((K, G), jnp.int32),
            pltpu.VMEM((K, G, F), jnp.float32),
            pltpu.VMEM((G, F), jnp.float32),
            pltpu.VMEM_SHARED((NP, F), jnp.float32),
            pltpu.SemaphoreType.DMA,
        ],
    )
    def agg(hs, src2, dst2, zrows, out0, out1, srcv, dstv, rows, zv, accum,
            sem):
        cid = lax.axis_index("c")
        sid = lax.axis_index("s")
        r0 = sid * RPT
        pltpu.sync_copy(zrows, zv)
        for z in range(RPT // G):
            pltpu.sync_copy(zv, accum.at[pl.ds(r0 + z * G, G)])
        plsc.subcore_barrier()

        w = cid * NS + sid

        def chunk(i, carry):
            base = w * GPW + i * K
            pltpu.sync_copy(src2.at[pl.ds(base, K)], srcv)
            pltpu.sync_copy(dst2.at[pl.ds(base, K)], dstv)
            cps = [pltpu.async_copy(hs.at[srcv.at[j]], rows.at[j], sem)
                   for j in range(K)]
            for cp in cps:
                cp.wait()
            for j in range(K):
                pltpu.sync_copy(rows.at[j], accum.at[dstv.at[j]], add=True)
            return carry

        lax.fori_loop(0, NCHUNK, chunk, 0)
        plsc.subcore_barrier()

        @pl.when(cid == 0)
        def _():
            pltpu.sync_copy(accum.at[pl.ds(r0, RPT)], out0.at[pl.ds(r0, RPT)])

        @pl.when(cid == 1)
        def _():
            pltpu.sync_copy(accum.at[pl.ds(r0, RPT)], out1.at[pl.ds(r0, RPT)])

    return agg


@functools.partial(
    pl.kernel,
    out_type=(jax.ShapeDtypeStruct((NP, 8), jnp.float32),
              jax.ShapeDtypeStruct((NP, 8), jnp.float32)),
    mesh=_mesh(),
    compiler_params=_SC_PARAMS,
    scratch_types=[
        pltpu.VMEM((K, G), jnp.int32),
        pltpu.VMEM((G, 8), jnp.float32),
        pltpu.VMEM((G, 8), jnp.float32),
        pltpu.VMEM_SHARED((NP, 8), jnp.float32),
    ],
)
def _degree(dst2, onesr, zrows, out0, out1, dstv, onev, zv, accum):
    cid = lax.axis_index("c")
    sid = lax.axis_index("s")
    r0 = sid * RPT
    pltpu.sync_copy(zrows, zv)
    for z in range(RPT // G):
        pltpu.sync_copy(zv, accum.at[pl.ds(r0 + z * G, G)])
    pltpu.sync_copy(onesr, onev)
    plsc.subcore_barrier()

    w = cid * NS + sid

    def chunk(i, carry):
        base = w * GPW + i * K
        pltpu.sync_copy(dst2.at[pl.ds(base, K)], dstv)
        for j in range(K):
            pltpu.sync_copy(onev, accum.at[dstv.at[j]], add=True)
        return carry

    lax.fori_loop(0, NCHUNK, chunk, 0)
    plsc.subcore_barrier()

    @pl.when(cid == 0)
    def _():
        pltpu.sync_copy(accum.at[pl.ds(r0, RPT)], out0.at[pl.ds(r0, RPT)])

    @pl.when(cid == 1)
    def _():
        pltpu.sync_copy(accum.at[pl.ds(r0, RPT)], out1.at[pl.ds(r0, RPT)])


_AGG32 = _make_agg(32)
_AGG64 = _make_agg(64)
_AGG8 = _make_agg(8)


def _row_spec(f):
    return pl.BlockSpec((BN, f), lambda i: (i, 0))


def _rep_spec(r, c):
    return pl.BlockSpec((r, c), lambda i: (0, 0))


def _tc_prep(deg0, deg1, xp, w1):
    def body(d0, d1, x, w, dv8, hs1):
        dv = lax.rsqrt(d0[:, :1] + d1[:, :1] + 1.0)
        dv8[...] = jnp.broadcast_to(dv, (BN, 8))
        hs1[...] = dv * jnp.dot(x[...], w[...],
                                preferred_element_type=jnp.float32)

    return pl.pallas_call(
        body,
        grid=(NBLK,),
        in_specs=[_row_spec(8), _row_spec(8), _row_spec(128),
                  _rep_spec(128, 32)],
        out_specs=(_row_spec(8), _row_spec(32)),
        out_shape=(jax.ShapeDtypeStruct((NP, 8), jnp.float32),
                   jax.ShapeDtypeStruct((NP, 32), jnp.float32)),
    )(deg0, deg1, xp, w1)


def _tc_post1(p0, p1, hs1, dv8, b1, g1, be1):
    def body(a0, a1, h, dv, b, g, be, out):
        d = dv[:, :1]
        t = d * (a0[...] + a1[...] + h[...]) + b[...]
        z = t * (g[...] * BNS) + be[...]
        out[...] = d * _elu(z)

    return pl.pallas_call(
        body,
        grid=(NBLK,),
        in_specs=[_row_spec(32), _row_spec(32), _row_spec(32), _row_spec(8),
                  _rep_spec(1, 32), _rep_spec(1, 32), _rep_spec(1, 32)],
        out_specs=_row_spec(32),
        out_shape=jax.ShapeDtypeStruct((NP, 32), jnp.float32),
    )(p0, p1, hs1, dv8, b1, g1, be1)


def _tc_post2(q0, q1, hs2, dv8, w2, b2, g2, be2):
    def body(a0, a1, h, dv, w, b, g, be, out):
        d = dv[:, :1]
        t = d * (a0[...] + a1[...] + h[...])
        u = jnp.dot(t, w[...], preferred_element_type=jnp.float32) + b[...]
        z = u * (g[...] * BNS) + be[...]
        out[...] = d * _elu(z)

    return pl.pallas_call(
        body,
        grid=(NBLK,),
        in_specs=[_row_spec(32), _row_spec(32), _row_spec(32), _row_spec(8),
                  _rep_spec(32, 64), _rep_spec(1, 64), _rep_spec(1, 64),
                  _rep_spec(1, 64)],
        out_specs=_row_spec(64),
        out_shape=jax.ShapeDtypeStruct((NP, 64), jnp.float32),
    )(q0, q1, hs2, dv8, w2, b2, g2, be2)


def _tc_post3(r0, r1, hs3, dv8, w3, b3, g3, be3, w4):
    def body(a0, a1, h, dv, w, b, g, be, w4r, out):
        d = dv[:, :1]
        t = d * (a0[...] + a1[...] + h[...])
        u = jnp.dot(t, w[...], preferred_element_type=jnp.float32) + b[...]
        z = u * (g[...] * BNS) + be[...]
        h3 = _elu(z)
        v = jnp.dot(h3, w4r[...], preferred_element_type=jnp.float32)
        out[...] = jnp.broadcast_to(d * v, (BN, 8))

    return pl.pallas_call(
        body,
        grid=(NBLK,),
        in_specs=[_row_spec(64), _row_spec(64), _row_spec(64), _row_spec(8),
                  _rep_spec(64, 128), _rep_spec(1, 128), _rep_spec(1, 128),
                  _rep_spec(1, 128), _rep_spec(128, 1)],
        out_specs=_row_spec(8),
        out_shape=jax.ShapeDtypeStruct((NP, 8), jnp.float32),
    )(r0, r1, hs3, dv8, w3, b3, g3, be3, w4)


def _tc_final(s0, s1, hs4, dv8, b4):
    def body(a0, a1, h, dv, b, out):
        t = dv[:, :1] * (a0[:, :1] + a1[:, :1] + h[:, :1]) + b[...]
        e = _elu(t)
        out[...] = 1.0 / (1.0 + jnp.exp(-e))

    return pl.pallas_call(
        body,
        grid=(NBLK,),
        in_specs=[_row_spec(8), _row_spec(8), _row_spec(8), _row_spec(8),
                  _rep_spec(1, 1)],
        out_specs=_row_spec(1),
        out_shape=jax.ShapeDtypeStruct((NP, 1), jnp.float32),
    )(s0, s1, hs4, dv8, b4)


def kernel(x, edge_index, W1, b1, W2, b2, W3, b3, W4, b4,
           g1, be1, g2, be2, g3, be3):
    src = edge_index[0]
    dst = edge_index[1]
    pad = jnp.full((E2 - E,), N, jnp.int32)
    src2 = jnp.concatenate([src, pad]).reshape(NG, G)
    dst2 = jnp.concatenate([dst, pad]).reshape(NG, G)
    xp = jnp.pad(x, ((0, NP - N), (0, 0)))

    z8 = jnp.zeros((G, 8), jnp.float32)
    z32 = jnp.zeros((G, 32), jnp.float32)
    z64 = jnp.zeros((G, 64), jnp.float32)
    o8 = jnp.ones((G, 8), jnp.float32)

    deg0, deg1 = _degree(dst2, o8, z8)
    dv8, hs1 = _tc_prep(deg0, deg1, xp, W1)

    p0, p1 = _AGG32(hs1, src2, dst2, z32)
    hs2 = _tc_post1(p0, p1, hs1, dv8, b1.reshape(1, 32), g1.reshape(1, 32),
                    be1.reshape(1, 32))

    q0, q1 = _AGG32(hs2, src2, dst2, z32)
    hs3 = _tc_post2(q0, q1, hs2, dv8, W2, b2.reshape(1, 64),
                    g2.reshape(1, 64), be2.reshape(1, 64))

    r0, r1 = _AGG64(hs3, src2, dst2, z64)
    hs4 = _tc_post3(r0, r1, hs3, dv8, W3, b3.reshape(1, 128),
                    g3.reshape(1, 128), be3.reshape(1, 128), W4)

    s0, s1 = _AGG8(hs4, src2, dst2, z8)
    out = _tc_final(s0, s1, hs4, dv8, b4.reshape(1, 1))
    return out[:N]
